# baseline (speedup 1.0000x reference)
import functools

import jax
import jax.numpy as jnp
from jax import lax
from jax.experimental import pallas as pl
from jax.experimental.pallas import tpu as pltpu, tpu_sc as plsc

_B = 16384
_D = 64

_NC = 2
_NS = 16
_NW = _NC * _NS
_BPW = _B // _NW
_G = 16
_NG = _BPW // _G

_mesh = plsc.VectorSubcoreMesh(core_axis_name="c", subcore_axis_name="s")


@functools.partial(
    pl.kernel,
    mesh=_mesh,
    out_type=(
        jax.ShapeDtypeStruct((_B, _D), jnp.float32),
        jax.ShapeDtypeStruct((_B, _D), jnp.float32),
        jax.ShapeDtypeStruct((_B, _D), jnp.float32),
    ),
    scratch_types=[
        pltpu.VMEM((_BPW,), jnp.int32),
        pltpu.VMEM((_BPW,), jnp.int32),
        pltpu.VMEM((_BPW,), jnp.int32),
        pltpu.VMEM((_BPW, _D), jnp.float32),
        pltpu.SemaphoreType.DMA,
        pltpu.SemaphoreType.DMA,
        pltpu.SemaphoreType.DMA,
        pltpu.SemaphoreType.DMA,
        pltpu.SemaphoreType.DMA,
        pltpu.SemaphoreType.DMA,
        pltpu.SemaphoreType.DMA,
        pltpu.SemaphoreType.DMA,
    ],
)
def _lookup(h_hbm, t_hbm, r_hbm, ent_hbm, rel_hbm,
            h_out, t_out, r_out,
            hi_v, ti_v, ri_v, stage_v, *sems):
    wid = lax.axis_index("s") * _NC + lax.axis_index("c")
    base = wid * _BPW
    sl = pl.ds(base, _BPW)
    pltpu.sync_copy(h_hbm.at[sl], hi_v)
    pltpu.sync_copy(t_hbm.at[sl], ti_v)
    pltpu.sync_copy(r_hbm.at[sl], ri_v)
    nsem = len(sems)

    def gather_one(idx_v, tbl, out):
        def body(g, carry):
            iv = idx_v[pl.ds(g * _G, _G)]
            for k in range(_G):
                row = g * _G + k
                pltpu.async_copy(tbl.at[pl.ds(iv[k], 1), :],
                                 stage_v.at[pl.ds(row, 1), :], sems[k % nsem])
            return carry

        lax.fori_loop(0, _NG, body, 0)

        for s in range(nsem):
            def drain(i, carry):
                pltpu.make_async_copy(tbl.at[pl.ds(0, 1), :],
                                      stage_v.at[pl.ds(0, 1), :], sems[s]).wait()
                return carry

            lax.fori_loop(0, _BPW * (_G // nsem) // _G, drain, 0)
        pltpu.sync_copy(stage_v, out.at[sl])

    gather_one(hi_v, ent_hbm, h_out)
    gather_one(ti_v, ent_hbm, t_out)
    gather_one(ri_v, rel_hbm, r_out)


def kernel(h, t, r, entity_table, relation_table):
    return _lookup(h.astype(jnp.int32), t.astype(jnp.int32),
                   r.astype(jnp.int32), entity_table, relation_table)
